# baseline (speedup 1.0000x reference)
import functools

import jax
import jax.numpy as jnp
from jax.experimental import pallas as pl
from jax.experimental.pallas import tpu as pltpu


def _pick_bm(n: int) -> int:
    for bm in (200, 128, 80, 40, 16, 8):
        if n % bm == 0:
            return bm
    return n


def _gcn_kernel(a_ref, x_ref, w1_ref, b1_ref, w2_ref, b2_ref,
                nw1_ref, nb1_ref, nw2t_ref, nb2_ref,
                out_ref, xw1_ref, h1_ref, colsum_ref, *, n_rows, bm):
    i = pl.program_id(0)
    nsteps = pl.num_programs(0)

    @pl.when(i == 0)
    def _init():
        xw1_ref[:] = jnp.dot(x_ref[:], w1_ref[:],
                             preferred_element_type=jnp.float32)
        colsum_ref[:] = jnp.zeros_like(colsum_ref)

    a_blk = a_ref[:]
    h1_blk = jnp.dot(a_blk, xw1_ref[:], preferred_element_type=jnp.float32)
    h1_ref[pl.ds(i * bm, bm), :] = jnp.maximum(h1_blk + b1_ref[:], 0.0)
    colsum_ref[:] += jnp.sum(a_blk, axis=0, keepdims=True)

    @pl.when(i == nsteps - 1)
    def _finish():
        v = jnp.dot(colsum_ref[:] * (1.0 / n_rows), h1_ref[:],
                    preferred_element_type=jnp.float32)
        hm = jnp.dot(v, w2_ref[:], preferred_element_type=jnp.float32) + b2_ref[:]
        g = jnp.maximum(
            jnp.dot(hm, nw1_ref[:], preferred_element_type=jnp.float32)
            + nb1_ref[:], 0.0)
        out_ref[:] = jnp.sum(g * nw2t_ref[:], axis=1, keepdims=True) + nb2_ref[:]


@jax.jit
def kernel(X, A, W1, b1, W2, b2, nW1, nb1, nW2, nb2):
    n, d = X.shape
    bm = _pick_bm(n)
    nsteps = n // bm

    b1r = b1.reshape(1, d)
    b2r = b2.reshape(1, d)
    nb1r = nb1.reshape(1, -1)
    nw2t = nW2.reshape(1, -1)
    nb2r = nb2.reshape(1, 1)

    grid_spec = pltpu.PrefetchScalarGridSpec(
        num_scalar_prefetch=0,
        grid=(nsteps,),
        in_specs=[
            pl.BlockSpec((bm, n), lambda i: (i, 0)),
            pl.BlockSpec((n, d), lambda i: (0, 0)),
            pl.BlockSpec(W1.shape, lambda i: (0, 0)),
            pl.BlockSpec(b1r.shape, lambda i: (0, 0)),
            pl.BlockSpec(W2.shape, lambda i: (0, 0)),
            pl.BlockSpec(b2r.shape, lambda i: (0, 0)),
            pl.BlockSpec(nW1.shape, lambda i: (0, 0)),
            pl.BlockSpec(nb1r.shape, lambda i: (0, 0)),
            pl.BlockSpec(nw2t.shape, lambda i: (0, 0)),
            pl.BlockSpec(nb2r.shape, lambda i: (0, 0)),
        ],
        out_specs=pl.BlockSpec((1, 1), lambda i: (0, 0)),
        scratch_shapes=[
            pltpu.VMEM((n, d), jnp.float32),
            pltpu.VMEM((n, d), jnp.float32),
            pltpu.VMEM((1, n), jnp.float32),
        ],
    )

    out = pl.pallas_call(
        functools.partial(_gcn_kernel, n_rows=n, bm=bm),
        grid_spec=grid_spec,
        out_shape=jax.ShapeDtypeStruct((1, 1), jnp.float32),
    )(A, X, W1, b1r, W2, b2r, nW1, nb1r, nw2t, nb2r)
    return out.reshape(1)


# speedup vs baseline: 1.0205x; 1.0205x over previous
import functools

import jax
import jax.numpy as jnp
from jax.experimental import pallas as pl
from jax.experimental.pallas import tpu as pltpu


def _pick_bm(n: int) -> int:
    for bm in (400, 256, 200, 128, 80, 40, 16, 8):
        if n % bm == 0:
            return bm
    return n


def _gcn_kernel(a_ref, x_ref, w1_ref, b1_ref, w2_ref, b2_ref,
                nw1_ref, nb1_ref, nw2t_ref, nb2_ref,
                out_ref, xw1_ref, h1_ref, colsum_ref, *, n_rows, bm):
    i = pl.program_id(0)
    nsteps = pl.num_programs(0)

    @pl.when(i == 0)
    def _init():
        xw1_ref[:] = jnp.dot(x_ref[:], w1_ref[:],
                             preferred_element_type=jnp.float32)
        colsum_ref[:] = jnp.zeros_like(colsum_ref)

    a_blk = a_ref[:]
    h1_blk = jnp.dot(a_blk, xw1_ref[:], preferred_element_type=jnp.float32)
    h1_ref[pl.ds(i * bm, bm), :] = jnp.maximum(h1_blk + b1_ref[:], 0.0)
    colsum_ref[:] += jnp.sum(a_blk, axis=0, keepdims=True)

    @pl.when(i == nsteps - 1)
    def _finish():
        v = jnp.dot(colsum_ref[:] * (1.0 / n_rows), h1_ref[:],
                    preferred_element_type=jnp.float32)
        hm = jnp.dot(v, w2_ref[:], preferred_element_type=jnp.float32) + b2_ref[:]
        g = jnp.maximum(
            jnp.dot(hm, nw1_ref[:], preferred_element_type=jnp.float32)
            + nb1_ref[:], 0.0)
        out_ref[:] = jnp.sum(g * nw2t_ref[:], axis=1, keepdims=True) + nb2_ref[:]


@jax.jit
def kernel(X, A, W1, b1, W2, b2, nW1, nb1, nW2, nb2):
    n, d = X.shape
    bm = _pick_bm(n)
    nsteps = n // bm

    b1r = b1.reshape(1, d)
    b2r = b2.reshape(1, d)
    nb1r = nb1.reshape(1, -1)
    nw2t = nW2.reshape(1, -1)
    nb2r = nb2.reshape(1, 1)

    grid_spec = pltpu.PrefetchScalarGridSpec(
        num_scalar_prefetch=0,
        grid=(nsteps,),
        in_specs=[
            pl.BlockSpec((bm, n), lambda i: (i, 0)),
            pl.BlockSpec((n, d), lambda i: (0, 0)),
            pl.BlockSpec(W1.shape, lambda i: (0, 0)),
            pl.BlockSpec(b1r.shape, lambda i: (0, 0)),
            pl.BlockSpec(W2.shape, lambda i: (0, 0)),
            pl.BlockSpec(b2r.shape, lambda i: (0, 0)),
            pl.BlockSpec(nW1.shape, lambda i: (0, 0)),
            pl.BlockSpec(nb1r.shape, lambda i: (0, 0)),
            pl.BlockSpec(nw2t.shape, lambda i: (0, 0)),
            pl.BlockSpec(nb2r.shape, lambda i: (0, 0)),
        ],
        out_specs=pl.BlockSpec((1, 1), lambda i: (0, 0)),
        scratch_shapes=[
            pltpu.VMEM((n, d), jnp.float32),
            pltpu.VMEM((n, d), jnp.float32),
            pltpu.VMEM((1, n), jnp.float32),
        ],
    )

    out = pl.pallas_call(
        functools.partial(_gcn_kernel, n_rows=n, bm=bm),
        grid_spec=grid_spec,
        out_shape=jax.ShapeDtypeStruct((1, 1), jnp.float32),
    )(A, X, W1, b1r, W2, b2r, nW1, nb1r, nw2t, nb2r)
    return out.reshape(1)
